# Initial kernel scaffold; baseline (speedup 1.0000x reference)
import jax
import jax.numpy as jnp
from jax.experimental import pallas as pl


def kernel(x, edge_index, W1, b1, W2, b2):
    raise NotImplementedError("write your pallas kernel here")


# speedup vs baseline: 12.7332x; 12.7332x over previous
import functools

import jax
import jax.numpy as jnp
from jax import lax
from jax.experimental import pallas as pl
from jax.experimental.pallas import tpu as pltpu
from jax.experimental.pallas import tpu_sc as plsc

N_NODES = 10000
N_EDGES = 320000
D_FEAT = 128
HIDDEN = 128
N_CLASSES = 64

NC, NS = 2, 16
NW = NC * NS
CH = 128
N_PAD = 10240
T = -(-N_EDGES // (NW * CH))
E_PAD = T * NW * CH
ROWS_PER_SUB = N_PAD // NS

_MESH = dict(mesh=plsc.VectorSubcoreMesh(core_axis_name="c", subcore_axis_name="s"))


def _zero_rows(buf, nrows, ncols16):
    z16 = jnp.zeros((16,), jnp.float32)

    def body(k, _):
        i = k // ncols16
        j = k % ncols16
        buf[i, pl.ds(j * 16, 16)] = z16
        return 0

    lax.fori_loop(0, nrows * ncols16, body, 0)


@functools.partial(
    pl.kernel,
    out_type=jax.ShapeDtypeStruct((NC, N_PAD, 16), jnp.float32),
    scratch_types=[
        pltpu.VMEM((T, CH), jnp.int32),
        pltpu.VMEM((CH, 16), jnp.float32),
        pltpu.VMEM((64, 16), jnp.float32),
        pltpu.VMEM_SHARED((N_PAD, 16), jnp.float32),
    ],
    **_MESH,
)
def _sc_hist(dst_hbm, out_hbm, dst_v, e0_v, z_v, acc_sh):
    c = lax.axis_index("c")
    s = lax.axis_index("s")
    wid = s * NC + c
    pltpu.sync_copy(dst_hbm.at[wid], dst_v)

    lane = lax.iota(jnp.int32, 16)
    one0 = jnp.where(lane == 0, jnp.float32(1), jnp.float32(0))

    def fill(i, _):
        e0_v[i, :] = one0
        return 0

    lax.fori_loop(0, CH, fill, 0)
    _zero_rows(z_v, 64, 1)

    def zinit(k, _):
        pltpu.sync_copy(z_v, acc_sh.at[pl.ds(s * ROWS_PER_SUB + k * 64, 64)])
        return 0

    lax.fori_loop(0, ROWS_PER_SUB // 64, zinit, 0)
    plsc.subcore_barrier()

    def body(j, _):
        pltpu.sync_copy(e0_v, acc_sh.at[dst_v.at[j]], add=True)
        return 0

    lax.fori_loop(0, T, body, 0)
    plsc.subcore_barrier()
    pltpu.sync_copy(
        acc_sh.at[pl.ds(s * ROWS_PER_SUB, ROWS_PER_SUB)],
        out_hbm.at[c, pl.ds(s * ROWS_PER_SUB, ROWS_PER_SUB)],
    )


@functools.partial(
    pl.kernel,
    out_type=jax.ShapeDtypeStruct((NC, N_PAD, HIDDEN), jnp.float32),
    scratch_types=[
        pltpu.VMEM((T, CH), jnp.int32),
        pltpu.VMEM((T, CH), jnp.int32),
        pltpu.VMEM((CH, HIDDEN), jnp.float32),
        pltpu.VMEM((64, HIDDEN), jnp.float32),
        pltpu.VMEM_SHARED((N_PAD, HIDDEN), jnp.float32),
        pltpu.SemaphoreType.DMA,
    ],
    **_MESH,
)
def _sc_hop(tab_hbm, src_hbm, dst_hbm, out_hbm, src_v, dst_v, gbuf, z_v, acc_sh, sem):
    c = lax.axis_index("c")
    s = lax.axis_index("s")
    wid = s * NC + c
    pltpu.sync_copy(src_hbm.at[wid], src_v)
    pltpu.sync_copy(dst_hbm.at[wid], dst_v)

    _zero_rows(z_v, 64, HIDDEN // 16)

    def zinit(k, _):
        pltpu.sync_copy(z_v, acc_sh.at[pl.ds(s * ROWS_PER_SUB + k * 64, 64)])
        return 0

    lax.fori_loop(0, ROWS_PER_SUB // 64, zinit, 0)
    plsc.subcore_barrier()

    def body(j, _):
        pltpu.async_copy(tab_hbm.at[src_v.at[j]], gbuf, sem).wait()
        pltpu.sync_copy(gbuf, acc_sh.at[dst_v.at[j]], add=True)
        return 0

    lax.fori_loop(0, T, body, 0)
    plsc.subcore_barrier()
    pltpu.sync_copy(
        acc_sh.at[pl.ds(s * ROWS_PER_SUB, ROWS_PER_SUB)],
        out_hbm.at[c, pl.ds(s * ROWS_PER_SUB, ROWS_PER_SUB)],
    )


_BLK = 1024


def _dinv_sq(hist_blk):
    deg = 1.0 + hist_blk[0, :, 0:1] + hist_blk[1, :, 0:1]
    return 1.0 / deg


def _prep_body(x_ref, w1_ref, hist_ref, z_ref):
    dinv = lax.rsqrt(1.0 / _dinv_sq(hist_ref[...]))
    y = lax.dot_general(
        x_ref[...], w1_ref[...], (((1,), (0,)), ((), ())),
        precision=lax.Precision.HIGHEST, preferred_element_type=jnp.float32)
    z_ref[...] = y * dinv


_prep = pl.pallas_call(
    _prep_body,
    grid=(N_PAD // _BLK,),
    in_specs=[
        pl.BlockSpec((_BLK, D_FEAT), lambda i: (i, 0)),
        pl.BlockSpec((D_FEAT, HIDDEN), lambda i: (0, 0)),
        pl.BlockSpec((NC, _BLK, 16), lambda i: (0, i, 0)),
    ],
    out_specs=pl.BlockSpec((_BLK, HIDDEN), lambda i: (i, 0)),
    out_shape=jax.ShapeDtypeStruct((N_PAD, HIDDEN), jnp.float32),
)


def _mid_body(s_ref, z_ref, hist_ref, out_ref):
    di2 = _dinv_sq(hist_ref[...])
    out_ref[...] = (s_ref[0] + s_ref[1] + z_ref[...]) * di2


_mid = pl.pallas_call(
    _mid_body,
    grid=(N_PAD // _BLK,),
    in_specs=[
        pl.BlockSpec((NC, _BLK, HIDDEN), lambda i: (0, i, 0)),
        pl.BlockSpec((_BLK, HIDDEN), lambda i: (i, 0)),
        pl.BlockSpec((NC, _BLK, 16), lambda i: (0, i, 0)),
    ],
    out_specs=pl.BlockSpec((_BLK, HIDDEN), lambda i: (i, 0)),
    out_shape=jax.ShapeDtypeStruct((N_PAD, HIDDEN), jnp.float32),
)


def _final_body(s_ref, z2_ref, hist_ref, b1_ref, w2_ref, b2_ref, logp_ref, hid_ref):
    di2 = _dinv_sq(hist_ref[...])
    dinv = lax.rsqrt(1.0 / di2)
    h2 = (s_ref[0] + s_ref[1] + z2_ref[...]) * dinv
    hidden = h2 + b1_ref[...]
    hid_ref[...] = hidden
    logits = lax.dot_general(
        hidden, w2_ref[...], (((1,), (0,)), ((), ())),
        precision=lax.Precision.HIGHEST, preferred_element_type=jnp.float32)
    logits = logits + b2_ref[...]
    m = jnp.max(logits, axis=1, keepdims=True)
    lse = m + jnp.log(jnp.sum(jnp.exp(logits - m), axis=1, keepdims=True))
    logp_ref[...] = logits - lse


_final = pl.pallas_call(
    _final_body,
    grid=(N_PAD // _BLK,),
    in_specs=[
        pl.BlockSpec((NC, _BLK, HIDDEN), lambda i: (0, i, 0)),
        pl.BlockSpec((_BLK, HIDDEN), lambda i: (i, 0)),
        pl.BlockSpec((NC, _BLK, 16), lambda i: (0, i, 0)),
        pl.BlockSpec((1, HIDDEN), lambda i: (0, 0)),
        pl.BlockSpec((HIDDEN, N_CLASSES), lambda i: (0, 0)),
        pl.BlockSpec((1, N_CLASSES), lambda i: (0, 0)),
    ],
    out_specs=[
        pl.BlockSpec((_BLK, N_CLASSES), lambda i: (i, 0)),
        pl.BlockSpec((_BLK, HIDDEN), lambda i: (i, 0)),
    ],
    out_shape=[
        jax.ShapeDtypeStruct((N_PAD, N_CLASSES), jnp.float32),
        jax.ShapeDtypeStruct((N_PAD, HIDDEN), jnp.float32),
    ],
)


def kernel(x, edge_index, W1, b1, W2, b2):
    x_pad = jnp.pad(x, ((0, N_PAD - N_NODES), (0, 0)))
    ei = edge_index.astype(jnp.int32)
    ei = jnp.pad(ei, ((0, 0), (0, E_PAD - N_EDGES)), constant_values=N_PAD - 1)
    src = ei[0].reshape(NW, T, CH)
    dst = ei[1].reshape(NW, T, CH)

    hist = _sc_hist(dst)
    z = _prep(x_pad, W1, hist)
    s1 = _sc_hop(z, src, dst)
    z2 = _mid(s1, z, hist)
    s2 = _sc_hop(z2, src, dst)
    logp, hidden = _final(s2, z2, hist, b1.reshape(1, -1), W2, b2.reshape(1, -1))
    return (logp[:N_NODES], hidden[:N_NODES])
